# baseline (speedup 1.0000x reference)
import functools

import jax
import jax.numpy as jnp
from jax import lax
from jax.experimental import pallas as pl
from jax.experimental.pallas import tpu as pltpu
from jax.experimental.pallas import tpu_sc as plsc

N_DOCS = 8
DOC_LEN = 256
M = DOC_LEN - 1
K = M - 1
TOTAL = N_DOCS * DOC_LEN


DOCS_PER_BLOCK = 8


def _shift_right(a, step):
    z = jnp.zeros(a.shape[:-1] + (step,), jnp.float32)
    return jnp.concatenate([z, a[..., : M - step]], axis=-1)


def _shift_left(a, step):
    z = jnp.zeros(a.shape[:-1] + (step,), jnp.float32)
    return jnp.concatenate([a[..., step:], z], axis=-1)


def _all_shifts(v, shift):
    rows = [v[None, :]]
    for k in range(1, 8):
        rows.append(shift(v[None, :], k))
    a = jnp.concatenate(rows, axis=0)
    for step in (8, 16, 32, 64, 128):
        a = jnp.concatenate([a, shift(a, step)], axis=0)
    return a


def _gate_tc_body(s_ref, fwd_out_ref, bwd_out_ref):
    u = (lax.broadcasted_iota(jnp.int32, (K, K), 0)
         <= lax.broadcasted_iota(jnp.int32, (K, K), 1)).astype(jnp.float32)

    def gate(t, base):
        x = (t - base[None, :]) * 100.0 + 5.0
        lsg = jnp.minimum(x, 0.0) - jnp.log1p(jnp.exp(-jnp.abs(x)))
        c = lax.dot_general(lsg, u, (((0,), (0,)), ((), ())),
                            preferred_element_type=jnp.float32,
                            precision=lax.Precision.HIGHEST)
        return jnp.exp(c)

    for i in range(DOCS_PER_BLOCK):
        s = s_ref[i, 0, :]
        fwd = s[:M]
        bwd = s[1:]
        t_fwd = _all_shifts(fwd, _shift_right)[:K]
        bwd1 = _shift_left(bwd[None, :], 1)[0]
        t_bwd = _all_shifts(bwd1, _shift_left)[:K]
        fwd_out_ref[i] = gate(t_fwd, fwd)
        bwd_out_ref[i] = gate(t_bwd, bwd)


def _gate_tc(s3d, interpret=False):
    nb = N_DOCS // DOCS_PER_BLOCK
    return pl.pallas_call(
        _gate_tc_body,
        grid=(nb,),
        in_specs=[pl.BlockSpec((DOCS_PER_BLOCK, 1, DOC_LEN),
                               lambda d: (d, 0, 0))],
        out_specs=[pl.BlockSpec((DOCS_PER_BLOCK, M, K), lambda d: (d, 0, 0)),
                   pl.BlockSpec((DOCS_PER_BLOCK, M, K), lambda d: (d, 0, 0))],
        out_shape=[jax.ShapeDtypeStruct((N_DOCS, M, K), jnp.float32),
                   jax.ShapeDtypeStruct((N_DOCS, M, K), jnp.float32)],
        interpret=interpret,
    )(s3d)


def _sc_gather(score, rep_idx):
    info = plsc.get_sparse_core_info()
    ncores = 1
    nw = ncores * info.num_subcores
    chunk = TOTAL // nw
    per_doc = nw // N_DOCS
    mesh = plsc.VectorSubcoreMesh(core_axis_name="c", subcore_axis_name="s",
                                  num_cores=ncores)

    @functools.partial(
        pl.kernel, mesh=mesh,
        out_type=jax.ShapeDtypeStruct((N_DOCS, 1, DOC_LEN), jnp.float32),
        compiler_params=pltpu.CompilerParams(needs_layout_passes=False),
        scratch_types=[pltpu.VMEM((chunk,), jnp.int32),
                       pltpu.VMEM((chunk,), jnp.float32),
                       pltpu.SemaphoreType.DMA],
    )
    def k(score_hbm, idx_hbm, out_hbm, idx_v, vals_v, sem):
        wid = lax.axis_index("s") * ncores + lax.axis_index("c")
        doc = wid // per_doc
        q = wid % per_doc
        pltpu.sync_copy(idx_hbm.at[doc, pl.ds(q * chunk, chunk)], idx_v)
        pltpu.async_copy(score_hbm.at[idx_v], vals_v, sem).wait()
        pltpu.sync_copy(vals_v, out_hbm.at[doc, 0, pl.ds(q * chunk, chunk)])

    return k(score, rep_idx)


def kernel(score, rep_srcs, rep_idx):
    del rep_srcs
    s3d = _sc_gather(score, rep_idx)
    return tuple(_gate_tc(s3d))


# speedup vs baseline: 1.0109x; 1.0109x over previous
import functools

import jax
import jax.numpy as jnp
from jax import lax
from jax.experimental import pallas as pl
from jax.experimental.pallas import tpu as pltpu
from jax.experimental.pallas import tpu_sc as plsc

N_DOCS = 8
DOC_LEN = 256
M = DOC_LEN - 1
K = M - 1
TOTAL = N_DOCS * DOC_LEN


DOCS_PER_BLOCK = 4


def _shift_right(a, step):
    z = jnp.zeros(a.shape[:-1] + (step,), jnp.float32)
    return jnp.concatenate([z, a[..., : M - step]], axis=-1)


def _shift_left(a, step):
    z = jnp.zeros(a.shape[:-1] + (step,), jnp.float32)
    return jnp.concatenate([a[..., step:], z], axis=-1)


def _all_shifts(v, shift):
    rows = [v[None, :]]
    for k in range(1, 8):
        rows.append(shift(v[None, :], k))
    a = jnp.concatenate(rows, axis=0)
    for step in (8, 16, 32, 64, 128):
        a = jnp.concatenate([a, shift(a, step)], axis=0)
    return a


def _gate_tc_body(s_ref, fwd_out_ref, bwd_out_ref):
    u = (lax.broadcasted_iota(jnp.int32, (K, K), 0)
         <= lax.broadcasted_iota(jnp.int32, (K, K), 1)).astype(jnp.float32)

    def gate(t, base):
        x = (t - base[None, :]) * 100.0 + 5.0
        lsg = jnp.minimum(x, 0.0) - jnp.log1p(jnp.exp(-jnp.abs(x)))
        c = lax.dot_general(lsg, u, (((0,), (0,)), ((), ())),
                            preferred_element_type=jnp.float32,
                            precision=lax.Precision.HIGHEST)
        return jnp.exp(c)

    for i in range(DOCS_PER_BLOCK):
        s = s_ref[i, 0, :]
        fwd = s[:M]
        bwd = s[1:]
        t_fwd = _all_shifts(fwd, _shift_right)[:K]
        bwd1 = _shift_left(bwd[None, :], 1)[0]
        t_bwd = _all_shifts(bwd1, _shift_left)[:K]
        fwd_out_ref[i] = gate(t_fwd, fwd)
        bwd_out_ref[i] = gate(t_bwd, bwd)


def _gate_tc(s3d, interpret=False):
    nb = N_DOCS // DOCS_PER_BLOCK
    return pl.pallas_call(
        _gate_tc_body,
        grid=(nb,),
        in_specs=[pl.BlockSpec((DOCS_PER_BLOCK, 1, DOC_LEN),
                               lambda d: (d, 0, 0))],
        out_specs=[pl.BlockSpec((DOCS_PER_BLOCK, M, K), lambda d: (d, 0, 0)),
                   pl.BlockSpec((DOCS_PER_BLOCK, M, K), lambda d: (d, 0, 0))],
        out_shape=[jax.ShapeDtypeStruct((N_DOCS, M, K), jnp.float32),
                   jax.ShapeDtypeStruct((N_DOCS, M, K), jnp.float32)],
        interpret=interpret,
    )(s3d)


def _sc_gather(score, rep_idx):
    info = plsc.get_sparse_core_info()
    ncores = 1
    nw = ncores * info.num_subcores
    chunk = TOTAL // nw
    per_doc = nw // N_DOCS
    mesh = plsc.VectorSubcoreMesh(core_axis_name="c", subcore_axis_name="s",
                                  num_cores=ncores)

    @functools.partial(
        pl.kernel, mesh=mesh,
        out_type=jax.ShapeDtypeStruct((N_DOCS, 1, DOC_LEN), jnp.float32),
        compiler_params=pltpu.CompilerParams(needs_layout_passes=False,
                                             skip_device_barrier=True),
        scratch_types=[pltpu.VMEM((chunk,), jnp.int32),
                       pltpu.VMEM((chunk,), jnp.float32),
                       pltpu.SemaphoreType.DMA],
    )
    def k(score_hbm, idx_hbm, out_hbm, idx_v, vals_v, sem):
        wid = lax.axis_index("s") * ncores + lax.axis_index("c")
        doc = wid // per_doc
        q = wid % per_doc
        pltpu.sync_copy(idx_hbm.at[doc, pl.ds(q * chunk, chunk)], idx_v)
        pltpu.async_copy(score_hbm.at[idx_v], vals_v, sem).wait()
        pltpu.sync_copy(vals_v, out_hbm.at[doc, 0, pl.ds(q * chunk, chunk)])

    return k(score, rep_idx)


def kernel(score, rep_srcs, rep_idx):
    del rep_srcs
    s3d = _sc_gather(score, rep_idx)
    return tuple(_gate_tc(s3d))


# speedup vs baseline: 1.0527x; 1.0414x over previous
import functools

import jax
import jax.numpy as jnp
from jax import lax
from jax.experimental import pallas as pl
from jax.experimental.pallas import tpu as pltpu
from jax.experimental.pallas import tpu_sc as plsc

N_DOCS = 8
DOC_LEN = 256
M = DOC_LEN - 1
K = M - 1
TOTAL = N_DOCS * DOC_LEN


DOCS_PER_BLOCK = 4


def _shift_right(a, step):
    z = jnp.zeros(a.shape[:-1] + (step,), jnp.float32)
    return jnp.concatenate([z, a[..., : M - step]], axis=-1)


def _shift_left(a, step):
    z = jnp.zeros(a.shape[:-1] + (step,), jnp.float32)
    return jnp.concatenate([a[..., step:], z], axis=-1)


def _all_shifts(v, shift):
    rows = [v[None, :]]
    for k in range(1, 8):
        rows.append(shift(v[None, :], k))
    a = jnp.concatenate(rows, axis=0)
    for step in (8, 16, 32, 64, 128):
        a = jnp.concatenate([a, shift(a, step)], axis=0)
    return a


def _gate_tc_body(s_ref, fwd_out_ref, bwd_out_ref):
    u = (lax.broadcasted_iota(jnp.int32, (K, K), 0)
         <= lax.broadcasted_iota(jnp.int32, (K, K), 1)).astype(jnp.bfloat16)
    log2e = 1.4426950408889634

    def gate(t, base):
        y = (t - base[None, :]) * (100.0 * log2e) + (5.0 * log2e)
        lsg2 = jnp.minimum(y, 0.0) - jnp.log2(1.0 + jnp.exp2(-jnp.abs(y)))
        h1 = lsg2.astype(jnp.bfloat16)
        r1 = lsg2 - h1.astype(jnp.float32)
        h2 = r1.astype(jnp.bfloat16)
        h3 = (r1 - h2.astype(jnp.float32)).astype(jnp.bfloat16)
        dn = (((0,), (0,)), ((), ()))
        c = (lax.dot_general(h1, u, dn, preferred_element_type=jnp.float32)
             + lax.dot_general(h2, u, dn, preferred_element_type=jnp.float32)
             + lax.dot_general(h3, u, dn, preferred_element_type=jnp.float32))
        return jnp.exp2(c)

    for i in range(DOCS_PER_BLOCK):
        s = s_ref[i, 0, :]
        fwd = s[:M]
        bwd = s[1:]
        t_fwd = _all_shifts(fwd, _shift_right)[:K]
        bwd1 = _shift_left(bwd[None, :], 1)[0]
        t_bwd = _all_shifts(bwd1, _shift_left)[:K]
        fwd_out_ref[i] = gate(t_fwd, fwd)
        bwd_out_ref[i] = gate(t_bwd, bwd)


def _gate_tc(s3d, interpret=False):
    nb = N_DOCS // DOCS_PER_BLOCK
    return pl.pallas_call(
        _gate_tc_body,
        grid=(nb,),
        in_specs=[pl.BlockSpec((DOCS_PER_BLOCK, 1, DOC_LEN),
                               lambda d: (d, 0, 0))],
        out_specs=[pl.BlockSpec((DOCS_PER_BLOCK, M, K), lambda d: (d, 0, 0)),
                   pl.BlockSpec((DOCS_PER_BLOCK, M, K), lambda d: (d, 0, 0))],
        out_shape=[jax.ShapeDtypeStruct((N_DOCS, M, K), jnp.float32),
                   jax.ShapeDtypeStruct((N_DOCS, M, K), jnp.float32)],
        interpret=interpret,
    )(s3d)


def _sc_gather(score, rep_idx):
    info = plsc.get_sparse_core_info()
    ncores = 1
    nw = ncores * info.num_subcores
    chunk = TOTAL // nw
    per_doc = nw // N_DOCS
    mesh = plsc.VectorSubcoreMesh(core_axis_name="c", subcore_axis_name="s",
                                  num_cores=ncores)

    @functools.partial(
        pl.kernel, mesh=mesh,
        out_type=jax.ShapeDtypeStruct((N_DOCS, 1, DOC_LEN), jnp.float32),
        compiler_params=pltpu.CompilerParams(needs_layout_passes=False),
        scratch_types=[pltpu.VMEM((chunk,), jnp.int32),
                       pltpu.VMEM((chunk,), jnp.float32),
                       pltpu.SemaphoreType.DMA],
    )
    def k(score_hbm, idx_hbm, out_hbm, idx_v, vals_v, sem):
        wid = lax.axis_index("s") * ncores + lax.axis_index("c")
        doc = wid // per_doc
        q = wid % per_doc
        pltpu.sync_copy(idx_hbm.at[doc, pl.ds(q * chunk, chunk)], idx_v)
        pltpu.async_copy(score_hbm.at[idx_v], vals_v, sem).wait()
        pltpu.sync_copy(vals_v, out_hbm.at[doc, 0, pl.ds(q * chunk, chunk)])

    return k(score, rep_idx)


def kernel(score, rep_srcs, rep_idx):
    del rep_srcs
    s3d = _sc_gather(score, rep_idx)
    return tuple(_gate_tc(s3d))


# speedup vs baseline: 1.0791x; 1.0251x over previous
import functools

import jax
import jax.numpy as jnp
from jax import lax
from jax.experimental import pallas as pl
from jax.experimental.pallas import tpu as pltpu
from jax.experimental.pallas import tpu_sc as plsc

N_DOCS = 8
DOC_LEN = 256
M = DOC_LEN - 1
K = M - 1
TOTAL = N_DOCS * DOC_LEN


DOCS_PER_BLOCK = 4


def _shift_right(a, step):
    z = jnp.zeros(a.shape[:-1] + (step,), jnp.float32)
    return jnp.concatenate([z, a[..., : M - step]], axis=-1)


def _shift_left(a, step):
    z = jnp.zeros(a.shape[:-1] + (step,), jnp.float32)
    return jnp.concatenate([a[..., step:], z], axis=-1)


def _all_shifts(v, shift):
    rows = [v[None, :]]
    for k in range(1, 8):
        rows.append(shift(v[None, :], k))
    a = jnp.concatenate(rows, axis=0)
    for step in (8, 16, 32, 64, 128):
        a = jnp.concatenate([a, shift(a, step)], axis=0)
    return a


def _gate_tc_body(s_ref, fwd_out_ref, bwd_out_ref):
    u = (lax.broadcasted_iota(jnp.int32, (K, K), 0)
         <= lax.broadcasted_iota(jnp.int32, (K, K), 1)).astype(jnp.bfloat16)
    log2e = 1.4426950408889634

    def gate(t, base):
        y = (t - base[None, :]) * (100.0 * log2e) + (5.0 * log2e)
        lsg2 = jnp.minimum(y, 0.0) - jnp.log2(1.0 + jnp.exp2(-jnp.abs(y)))
        h1 = lsg2.astype(jnp.bfloat16)
        h2 = (lsg2 - h1.astype(jnp.float32)).astype(jnp.bfloat16)
        dn = (((0,), (0,)), ((), ()))
        c = (lax.dot_general(h1, u, dn, preferred_element_type=jnp.float32)
             + lax.dot_general(h2, u, dn, preferred_element_type=jnp.float32))
        return jnp.exp2(c)

    for i in range(DOCS_PER_BLOCK):
        s = s_ref[i, 0, :]
        fwd = s[:M]
        bwd = s[1:]
        t_fwd = _all_shifts(fwd, _shift_right)[:K]
        bwd1 = _shift_left(bwd[None, :], 1)[0]
        t_bwd = _all_shifts(bwd1, _shift_left)[:K]
        fwd_out_ref[i] = gate(t_fwd, fwd)
        bwd_out_ref[i] = gate(t_bwd, bwd)


def _gate_tc(s3d, interpret=False):
    nb = N_DOCS // DOCS_PER_BLOCK
    return pl.pallas_call(
        _gate_tc_body,
        grid=(nb,),
        in_specs=[pl.BlockSpec((DOCS_PER_BLOCK, 1, DOC_LEN),
                               lambda d: (d, 0, 0))],
        out_specs=[pl.BlockSpec((DOCS_PER_BLOCK, M, K), lambda d: (d, 0, 0)),
                   pl.BlockSpec((DOCS_PER_BLOCK, M, K), lambda d: (d, 0, 0))],
        out_shape=[jax.ShapeDtypeStruct((N_DOCS, M, K), jnp.float32),
                   jax.ShapeDtypeStruct((N_DOCS, M, K), jnp.float32)],
        interpret=interpret,
    )(s3d)


def _sc_gather(score, rep_idx):
    info = plsc.get_sparse_core_info()
    ncores = 1
    nw = ncores * info.num_subcores
    chunk = TOTAL // nw
    per_doc = nw // N_DOCS
    mesh = plsc.VectorSubcoreMesh(core_axis_name="c", subcore_axis_name="s",
                                  num_cores=ncores)

    @functools.partial(
        pl.kernel, mesh=mesh,
        out_type=jax.ShapeDtypeStruct((N_DOCS, 1, DOC_LEN), jnp.float32),
        compiler_params=pltpu.CompilerParams(needs_layout_passes=False),
        scratch_types=[pltpu.VMEM((chunk,), jnp.int32),
                       pltpu.VMEM((chunk,), jnp.float32),
                       pltpu.SemaphoreType.DMA],
    )
    def k(score_hbm, idx_hbm, out_hbm, idx_v, vals_v, sem):
        wid = lax.axis_index("s") * ncores + lax.axis_index("c")
        doc = wid // per_doc
        q = wid % per_doc
        pltpu.sync_copy(idx_hbm.at[doc, pl.ds(q * chunk, chunk)], idx_v)
        pltpu.async_copy(score_hbm.at[idx_v], vals_v, sem).wait()
        pltpu.sync_copy(vals_v, out_hbm.at[doc, 0, pl.ds(q * chunk, chunk)])

    return k(score, rep_idx)


def kernel(score, rep_srcs, rep_idx):
    del rep_srcs
    s3d = _sc_gather(score, rep_idx)
    return tuple(_gate_tc(s3d))
